# baseline (speedup 1.0000x reference)
import functools

import jax
import jax.numpy as jnp
from jax import lax
from jax.experimental import pallas as pl
from jax.experimental.pallas import tpu as pltpu
from jax.experimental.pallas import tpu_sc as plsc

B = 16384
D = 16
MU = 3.5
NC = 2
NS = 16
NW = NC * NS
BPW = B // NW
CHUNK = 128
NCH = BPW // CHUNK


def _permute(x, pm):
    dnums = lax.GatherDimensionNumbers(
        offset_dims=(), collapsed_slice_dims=(0,), start_index_map=(0,))
    return lax.gather(x, pm[:, None], dnums, (1,),
                      mode=lax.GatherScatterMode.PROMISE_IN_BOUNDS)


def _make_sc_kernel():
    mesh = plsc.VectorSubcoreMesh(core_axis_name="c", subcore_axis_name="s")

    @functools.partial(
        pl.kernel,
        mesh=mesh,
        out_type=jax.ShapeDtypeStruct((B,), jnp.float32),
        compiler_params=pltpu.CompilerParams(use_tc_tiling_on_sc=False),
        scratch_types=[
            pltpu.VMEM((NCH, CHUNK), jnp.int32),
            pltpu.VMEM((NCH, CHUNK), jnp.int32),
            pltpu.VMEM((BPW, D), jnp.float32),
            pltpu.VMEM((BPW, D), jnp.float32),
            pltpu.VMEM((BPW,), jnp.float32),
            pltpu.VMEM((BPW,), jnp.float32),
            pltpu.VMEM((BPW,), jnp.float32),
            pltpu.SemaphoreType.DMA,
        ],
    )
    def sc_kernel(uidx_hbm, iidx_hbm, uw_hbm, iw_hbm, ubw_hbm, ibw_hbm,
                  out_hbm, uidx_v, iidx_v, urows, irows, ub_v, ib_v,
                  out_v, sem):
        wid = lax.axis_index("s") * NC + lax.axis_index("c")

        pltpu.sync_copy(uidx_hbm.at[wid], uidx_v)
        pltpu.sync_copy(iidx_hbm.at[wid], iidx_v)

        copies = []
        for j in range(NCH):
            sl = pl.ds(j * CHUNK, CHUNK)
            copies.append(pltpu.async_copy(uw_hbm.at[uidx_v.at[j]], urows.at[sl], sem))
            copies.append(pltpu.async_copy(iw_hbm.at[iidx_v.at[j]], irows.at[sl], sem))
            copies.append(pltpu.async_copy(ubw_hbm.at[uidx_v.at[j]], ub_v.at[sl], sem))
            copies.append(pltpu.async_copy(ibw_hbm.at[iidx_v.at[j]], ib_v.at[sl], sem))
        for c in copies:
            c.wait()

        lane = lax.broadcasted_iota(jnp.int32, (16,), 0)
        perms = [lane ^ k for k in (8, 4, 2, 1)]
        for s in range(BPW // 16):
            acc = (ub_v[pl.ds(s * 16, 16)]
                   + ib_v[pl.ds(s * 16, 16)]
                   + jnp.float32(MU))
            for r in range(16):
                row = s * 16 + r
                p = urows[row] * irows[row]
                for pm in perms:
                    p = p + _permute(p, pm)
                acc = jnp.where(lane == r, acc + p, acc)
            out_v[pl.ds(s * 16, 16)] = acc

        pltpu.sync_copy(out_v, out_hbm.at[pl.ds(wid * BPW, BPW)])

    return sc_kernel


_sc_kernel = _make_sc_kernel()


@jax.jit
def kernel(user_idx, item_idx, embed_user_w, embed_item_w, user_bias_w, item_bias_w):
    uidx3 = user_idx.astype(jnp.int32).reshape(NW, NCH, CHUNK)
    iidx3 = item_idx.astype(jnp.int32).reshape(NW, NCH, CHUNK)
    return _sc_kernel(uidx3, iidx3, embed_user_w, embed_item_w,
                      user_bias_w.reshape(-1), item_bias_w.reshape(-1))
